# baseline (speedup 1.0000x reference)
import functools
import math

import jax
import jax.numpy as jnp
from jax import lax
from jax.experimental import pallas as pl
from jax.experimental.pallas import tpu as pltpu
from jax.experimental.pallas import tpu_sc as plsc

D_MODEL = 768
PRETR_VOCAB = 100000
ADD_VOCAB = 1000
B = 4
S = 8192
N = B * S
SCALE = math.sqrt(D_MODEL)

NUM_CORES = 2
NUM_SUBCORES = 16
NW = NUM_CORES * NUM_SUBCORES
PER_W = N // NW
CHUNK = 64
N_CHUNKS = PER_W // CHUNK
LANES = 16
SEGS = D_MODEL // LANES


def _embed_body(x_hbm, pretr_hbm, add_hbm, out_hbm,
                idx_v, idx_p, idx_a, rows_p, rows_a, sem):
    wid = lax.axis_index("s") * NUM_CORES + lax.axis_index("c")
    base = wid * PER_W
    pltpu.sync_copy(x_hbm.at[pl.ds(base, PER_W)], idx_v)

    def chunk_body(c, carry):
        for g in range(CHUNK // LANES):
            iv = idx_v[pl.ds(c * CHUNK + g * LANES, LANES)]
            idx_p[pl.ds(g * LANES, LANES)] = jnp.minimum(iv, PRETR_VOCAB - 1)
            idx_a[pl.ds(g * LANES, LANES)] = jnp.maximum(iv - PRETR_VOCAB, 0)

        cp = pltpu.async_copy(pretr_hbm.at[idx_p], rows_p, sem)
        ca = pltpu.async_copy(add_hbm.at[idx_a], rows_a, sem)
        cp.wait()
        ca.wait()

        def group_body(g, gcarry):
            iv = idx_v[pl.ds(c * CHUNK + g * LANES, LANES)]
            for r in range(LANES):
                sv = iv[r]
                mp = jnp.where(sv < PRETR_VOCAB,
                               jnp.float32(SCALE), jnp.float32(0.0))
                ma = jnp.float32(SCALE) - mp
                mpv = jnp.full((LANES,), mp, jnp.float32)
                mav = jnp.full((LANES,), ma, jnp.float32)
                row = g * LANES + r

                def seg_body(j, scarry):
                    p = rows_p[row, pl.ds(j * LANES, LANES)]
                    a = rows_a[row, pl.ds(j * LANES, LANES)]
                    rows_p[row, pl.ds(j * LANES, LANES)] = p * mpv + a * mav
                    return scarry

                lax.fori_loop(0, SEGS, seg_body, 0)
            return gcarry

        lax.fori_loop(0, CHUNK // LANES, group_body, 0)
        pltpu.sync_copy(rows_p, out_hbm.at[pl.ds(base + c * CHUNK, CHUNK)])
        return carry

    lax.fori_loop(0, N_CHUNKS, chunk_body, 0)


_mesh = plsc.VectorSubcoreMesh(core_axis_name="c", subcore_axis_name="s")

_embed_call = functools.partial(
    pl.kernel,
    mesh=_mesh,
    out_type=jax.ShapeDtypeStruct((N, D_MODEL), jnp.float32),
    scratch_types=[
        pltpu.VMEM((PER_W,), jnp.int32),
        pltpu.VMEM((CHUNK,), jnp.int32),
        pltpu.VMEM((CHUNK,), jnp.int32),
        pltpu.VMEM((CHUNK, D_MODEL), jnp.float32),
        pltpu.VMEM((CHUNK, D_MODEL), jnp.float32),
        pltpu.SemaphoreType.DMA,
    ],
)(_embed_body)


@jax.jit
def kernel(x, pretrained, add_tokens):
    out = _embed_call(x.reshape(N), pretrained, add_tokens)
    return out.reshape(B, S, D_MODEL)


# speedup vs baseline: 8.8943x; 8.8943x over previous
import functools
import math

import jax
import jax.numpy as jnp
from jax import lax
from jax.experimental import pallas as pl
from jax.experimental.pallas import tpu as pltpu
from jax.experimental.pallas import tpu_sc as plsc

D_MODEL = 768
PRETR_VOCAB = 100000
ADD_VOCAB = 1000
B = 4
S = 8192
N = B * S
SCALE = math.sqrt(D_MODEL)

NUM_CORES = 2
NUM_SUBCORES = 16
NW = NUM_CORES * NUM_SUBCORES
PER_W = N // NW
CHUNK = 64
N_CHUNKS = PER_W // CHUNK
LANES = 16
SEGS = D_MODEL // LANES


def _embed_body(x_hbm, pretr_hbm, add_hbm, out_hbm,
                idx_v, idx_p, rows, sem):
    wid = lax.axis_index("s") * NUM_CORES + lax.axis_index("c")
    base = wid * PER_W
    pltpu.sync_copy(x_hbm.at[pl.ds(base, PER_W)], idx_v)

    def chunk_body(c, carry):
        for g in range(CHUNK // LANES):
            iv = idx_v[pl.ds(c * CHUNK + g * LANES, LANES)]
            idx_p[pl.ds(g * LANES, LANES)] = jnp.minimum(iv, PRETR_VOCAB - 1)

        pltpu.async_copy(pretr_hbm.at[idx_p], rows, sem).wait()

        for g in range(CHUNK // LANES):
            iv = idx_v[pl.ds(c * CHUNK + g * LANES, LANES)]
            for r in range(LANES):
                sv = iv[r]

                @pl.when(sv >= PRETR_VOCAB)
                def _():
                    pltpu.sync_copy(
                        add_hbm.at[pl.ds(sv - PRETR_VOCAB, 1)],
                        rows.at[pl.ds(g * LANES + r, 1)])

        def row_body(r, rcarry):
            for j in range(SEGS):
                rows[r, pl.ds(j * LANES, LANES)] = (
                    rows[r, pl.ds(j * LANES, LANES)] * jnp.float32(SCALE))
            return rcarry

        lax.fori_loop(0, CHUNK, row_body, 0)
        pltpu.sync_copy(rows, out_hbm.at[pl.ds(base + c * CHUNK, CHUNK)])
        return carry

    lax.fori_loop(0, N_CHUNKS, chunk_body, 0)


_mesh = plsc.VectorSubcoreMesh(core_axis_name="c", subcore_axis_name="s")

_embed_call = functools.partial(
    pl.kernel,
    mesh=_mesh,
    out_type=jax.ShapeDtypeStruct((N, D_MODEL), jnp.float32),
    scratch_types=[
        pltpu.VMEM((PER_W,), jnp.int32),
        pltpu.VMEM((CHUNK,), jnp.int32),
        pltpu.VMEM((CHUNK, D_MODEL), jnp.float32),
        pltpu.SemaphoreType.DMA,
    ],
)(_embed_body)


@jax.jit
def kernel(x, pretrained, add_tokens):
    out = _embed_call(x.reshape(N), pretrained, add_tokens)
    return out.reshape(B, S, D_MODEL)


# speedup vs baseline: 11.5480x; 1.2984x over previous
import functools
import math

import jax
import jax.numpy as jnp
from jax import lax
from jax.experimental import pallas as pl
from jax.experimental.pallas import tpu as pltpu
from jax.experimental.pallas import tpu_sc as plsc

D_MODEL = 768
PRETR_VOCAB = 100000
ADD_VOCAB = 1000
B = 4
S = 8192
N = B * S
SCALE = math.sqrt(D_MODEL)

NUM_CORES = 2
NUM_SUBCORES = 16
NW = NUM_CORES * NUM_SUBCORES
PER_W = N // NW
CHUNK = 32
N_CHUNKS = PER_W // CHUNK
NBUF = 4
LANES = 16
SEGS = D_MODEL // LANES


def _embed_body(x_hbm, pretr_hbm, add_hbm, out_hbm,
                idx_v,
                ip0, ip1, ip2, ip3,
                rb0, rb1, rb2, rb3,
                sg0, sg1, sg2, sg3,
                so0, so1, so2, so3):
    ip = [ip0, ip1, ip2, ip3]
    rb = [rb0, rb1, rb2, rb3]
    sg = [sg0, sg1, sg2, sg3]
    so = [so0, so1, so2, so3]

    wid = lax.axis_index("s") * NUM_CORES + lax.axis_index("c")
    base = wid * PER_W
    pltpu.sync_copy(x_hbm.at[pl.ds(base, PER_W)], idx_v)

    def prep(c, b):
        for g in range(CHUNK // LANES):
            iv = idx_v[pl.ds(c * CHUNK + g * LANES, LANES)]
            ip[b][pl.ds(g * LANES, LANES)] = jnp.minimum(iv, PRETR_VOCAB - 1)

    prep(0, 0)
    pltpu.async_copy(pretr_hbm.at[ip[0]], rb[0], sg[0])
    prep(1, 1)
    pltpu.async_copy(pretr_hbm.at[ip[1]], rb[1], sg[1])

    def iter_body(gi, carry):
        for b in range(NBUF):
            c = gi * NBUF + b
            pltpu.make_async_copy(pretr_hbm.at[ip[b]], rb[b], sg[b]).wait()

            for g in range(CHUNK // LANES):
                iv = idx_v[pl.ds(c * CHUNK + g * LANES, LANES)]
                for r in range(LANES):
                    sv = iv[r]

                    @pl.when(sv >= PRETR_VOCAB)
                    def _():
                        pltpu.sync_copy(
                            add_hbm.at[pl.ds(sv - PRETR_VOCAB, 1)],
                            rb[b].at[pl.ds(g * LANES + r, 1)])

            def row_body(r, rcarry):
                for j in range(SEGS):
                    rb[b][r, pl.ds(j * LANES, LANES)] = (
                        rb[b][r, pl.ds(j * LANES, LANES)] * jnp.float32(SCALE))
                return rcarry

            lax.fori_loop(0, CHUNK, row_body, 0)
            pltpu.async_copy(
                rb[b], out_hbm.at[pl.ds(base + c * CHUNK, CHUNK)], so[b])

            b2 = (b + 2) % NBUF

            @pl.when(c + 2 < N_CHUNKS)
            def _():
                prep(c + 2, b2)

                @pl.when(c >= 2)
                def _():
                    pltpu.make_async_copy(
                        rb[b2], out_hbm.at[pl.ds(base, CHUNK)],
                        so[b2]).wait()

                pltpu.async_copy(pretr_hbm.at[ip[b2]], rb[b2], sg[b2])
        return carry

    lax.fori_loop(0, N_CHUNKS // NBUF, iter_body, 0)

    for b in range(NBUF):
        pltpu.make_async_copy(
            rb[b], out_hbm.at[pl.ds(base, CHUNK)], so[b]).wait()


_mesh = plsc.VectorSubcoreMesh(core_axis_name="c", subcore_axis_name="s")

_embed_call = functools.partial(
    pl.kernel,
    mesh=_mesh,
    out_type=jax.ShapeDtypeStruct((N, D_MODEL), jnp.float32),
    scratch_types=(
        [pltpu.VMEM((PER_W,), jnp.int32)]
        + [pltpu.VMEM((CHUNK,), jnp.int32) for _ in range(NBUF)]
        + [pltpu.VMEM((CHUNK, D_MODEL), jnp.float32) for _ in range(NBUF)]
        + [pltpu.SemaphoreType.DMA for _ in range(2 * NBUF)]
    ),
)(_embed_body)


@jax.jit
def kernel(x, pretrained, add_tokens):
    out = _embed_call(x.reshape(N), pretrained, add_tokens)
    return out.reshape(B, S, D_MODEL)
